# baseline (speedup 1.0000x reference)
import functools

import jax
import jax.numpy as jnp
from jax import lax
from jax.experimental import pallas as pl
from jax.experimental.pallas import tpu as pltpu
from jax.experimental.pallas import tpu_sc as plsc

_NUM_CLASSES = 100000
_D = 128
_B = 4096
_LAMBDA = 0.003

_NC = 2
_NS = 16
_L = 16
_NW = _NC * _NS
_BPW = _B // _NW
_COLS = _D // _L
_C = 32
_NCH = _BPW // _C

_mesh = plsc.VectorSubcoreMesh(core_axis_name="c", subcore_axis_name="s")


@functools.partial(
    pl.kernel,
    out_type=jax.ShapeDtypeStruct((_NW, _L), jnp.float32),
    mesh=_mesh,
    scratch_types=[
        pltpu.VMEM((_BPW,), jnp.int32),
        pltpu.VMEM((_C, _D), jnp.float32),
        pltpu.VMEM((_C, _D), jnp.float32),
        pltpu.VMEM((_C, _D), jnp.float32),
        pltpu.VMEM((_C, _D), jnp.float32),
        pltpu.VMEM((_BPW, _D), jnp.float32),
        pltpu.VMEM((_L,), jnp.float32),
        pltpu.SemaphoreType.DMA,
        pltpu.SemaphoreType.DMA,
        pltpu.SemaphoreType.DMA,
        pltpu.SemaphoreType.DMA,
        pltpu.SemaphoreType.DMA,
    ],
)
def _sc_partial_sums(feat_hbm, lab_hbm, cent_hbm, out_hbm,
                     idx_v, rows0, rows1, rows2, rows3, feat_v, acc_v,
                     sg0, sg1, sg2, sg3, sf):
    wid = lax.axis_index("c") * _NS + lax.axis_index("s")
    base = wid * _BPW

    rows_bufs = (rows0, rows1, rows2, rows3)
    g_sems = (sg0, sg1, sg2, sg3)

    cp_f = pltpu.async_copy(feat_hbm.at[pl.ds(base, _BPW)], feat_v, sf)
    pltpu.sync_copy(lab_hbm.at[pl.ds(base, _BPW)], idx_v)

    copies = [
        pltpu.async_copy(cent_hbm.at[idx_v.at[pl.ds(k * _C, _C)]],
                         rows_bufs[k], g_sems[k])
        for k in range(_NCH)
    ]
    cp_f.wait()
    acc = jnp.zeros((_L,), jnp.float32)
    for k in range(_NCH):
        copies[k].wait()
        rbuf = rows_bufs[k]
        off = k * _C

        def chunk_body(i, acc, rbuf=rbuf, off=off):
            for j in range(_COLS):
                f = feat_v[off + i, pl.ds(j * _L, _L)]
                c = rbuf[i, pl.ds(j * _L, _L)]
                d = f - c
                acc = acc + d * d
            return acc

        acc = plsc.parallel_loop(0, _C, carry=acc, unroll=2)(chunk_body)

    acc_v[...] = acc
    pltpu.sync_copy(acc_v, out_hbm.at[wid])


def _tc_finish(p_ref, o_ref):
    o_ref[0, 0] = jnp.sum(p_ref[...]) * (_LAMBDA / _B)


_finish_call = pl.pallas_call(
    _tc_finish,
    out_shape=jax.ShapeDtypeStruct((1, 1), jnp.float32),
    out_specs=pl.BlockSpec(memory_space=pltpu.SMEM),
)


@jax.jit
def kernel(features, labels, centers):
    partials = _sc_partial_sums(features, labels.astype(jnp.int32), centers)
    return _finish_call(partials)[0, 0]


# speedup vs baseline: 1.0123x; 1.0123x over previous
import functools

import jax
import jax.numpy as jnp
from jax import lax
from jax.experimental import pallas as pl
from jax.experimental.pallas import tpu as pltpu
from jax.experimental.pallas import tpu_sc as plsc

_NUM_CLASSES = 100000
_D = 128
_B = 4096
_LAMBDA = 0.003

_NC = 2
_NS = 16
_L = 16
_NW = _NC * _NS
_BPW = _B // _NW
_COLS = _D // _L
_C = 64
_NCH = _BPW // _C

_mesh = plsc.VectorSubcoreMesh(core_axis_name="c", subcore_axis_name="s")


@functools.partial(
    pl.kernel,
    out_type=jax.ShapeDtypeStruct((_NW, _L), jnp.float32),
    mesh=_mesh,
    scratch_types=[
        pltpu.VMEM((_BPW,), jnp.int32),
        pltpu.VMEM((_C, _D), jnp.float32),
        pltpu.VMEM((_C, _D), jnp.float32),
        pltpu.VMEM((_BPW, _D), jnp.float32),
        pltpu.VMEM((_L,), jnp.float32),
        pltpu.SemaphoreType.DMA,
        pltpu.SemaphoreType.DMA,
        pltpu.SemaphoreType.DMA,
    ],
)
def _sc_partial_sums(feat_hbm, lab_hbm, cent_hbm, out_hbm,
                     idx_v, rows0, rows1, feat_v, acc_v,
                     sg0, sg1, sf):
    wid = lax.axis_index("c") * _NS + lax.axis_index("s")
    base = wid * _BPW

    rows_bufs = (rows0, rows1)
    g_sems = (sg0, sg1)

    cp_f = pltpu.async_copy(feat_hbm.at[pl.ds(base, _BPW)], feat_v, sf)
    pltpu.sync_copy(lab_hbm.at[pl.ds(base, _BPW)], idx_v)

    copies = [
        pltpu.async_copy(cent_hbm.at[idx_v.at[pl.ds(k * _C, _C)]],
                         rows_bufs[k], g_sems[k])
        for k in range(_NCH)
    ]
    cp_f.wait()
    acc = jnp.zeros((_L,), jnp.float32)
    for k in range(_NCH):
        copies[k].wait()
        rbuf = rows_bufs[k]
        off = k * _C

        def chunk_body(i, acc, rbuf=rbuf, off=off):
            for j in range(_COLS):
                f = feat_v[off + i, pl.ds(j * _L, _L)]
                c = rbuf[i, pl.ds(j * _L, _L)]
                d = f - c
                acc = acc + d * d
            return acc

        acc = plsc.parallel_loop(0, _C, carry=acc, unroll=2)(chunk_body)

    acc_v[...] = acc
    pltpu.sync_copy(acc_v, out_hbm.at[wid])


def _tc_finish(p_ref, o_ref):
    o_ref[0, 0] = jnp.sum(p_ref[...]) * (_LAMBDA / _B)


_finish_call = pl.pallas_call(
    _tc_finish,
    out_shape=jax.ShapeDtypeStruct((1, 1), jnp.float32),
    out_specs=pl.BlockSpec(memory_space=pltpu.SMEM),
)


@jax.jit
def kernel(features, labels, centers):
    partials = _sc_partial_sums(features, labels.astype(jnp.int32), centers)
    return _finish_call(partials)[0, 0]


# speedup vs baseline: 1.0203x; 1.0079x over previous
import functools

import jax
import jax.numpy as jnp
from jax import lax
from jax.experimental import pallas as pl
from jax.experimental.pallas import tpu as pltpu
from jax.experimental.pallas import tpu_sc as plsc

_NUM_CLASSES = 100000
_D = 128
_B = 4096
_LAMBDA = 0.003

_NC = 2
_NS = 16
_L = 16
_NW = _NC * _NS
_BPW = _B // _NW
_COLS = _D // _L
_C = 64
_NCH = _BPW // _C

_mesh = plsc.VectorSubcoreMesh(core_axis_name="c", subcore_axis_name="s")


@functools.partial(
    pl.kernel,
    out_type=jax.ShapeDtypeStruct((_NW // 8, 8 * _L), jnp.float32),
    mesh=_mesh,
    scratch_types=[
        pltpu.VMEM((_BPW,), jnp.int32),
        pltpu.VMEM((_C, _D), jnp.float32),
        pltpu.VMEM((_C, _D), jnp.float32),
        pltpu.VMEM((_BPW, _D), jnp.float32),
        pltpu.VMEM((_L,), jnp.float32),
        pltpu.SemaphoreType.DMA,
        pltpu.SemaphoreType.DMA,
        pltpu.SemaphoreType.DMA,
    ],
)
def _sc_partial_sums(feat_hbm, lab_hbm, cent_hbm, out_hbm,
                     idx_v, rows0, rows1, feat_v, acc_v,
                     sg0, sg1, sf):
    wid = lax.axis_index("c") * _NS + lax.axis_index("s")
    base = wid * _BPW

    rows_bufs = (rows0, rows1)
    g_sems = (sg0, sg1)

    cp_f = pltpu.async_copy(feat_hbm.at[pl.ds(base, _BPW)], feat_v, sf)
    pltpu.sync_copy(lab_hbm.at[pl.ds(base, _BPW)], idx_v)

    copies = [
        pltpu.async_copy(cent_hbm.at[idx_v.at[pl.ds(k * _C, _C)]],
                         rows_bufs[k], g_sems[k])
        for k in range(_NCH)
    ]
    cp_f.wait()
    acc = jnp.zeros((_L,), jnp.float32)
    for k in range(_NCH):
        copies[k].wait()
        rbuf = rows_bufs[k]
        off = k * _C

        def chunk_body(i, acc, rbuf=rbuf, off=off):
            for j in range(_COLS):
                f = feat_v[off + i, pl.ds(j * _L, _L)]
                c = rbuf[i, pl.ds(j * _L, _L)]
                d = f - c
                acc = acc + d * d
            return acc

        acc = plsc.parallel_loop(0, _C, carry=acc, unroll=2)(chunk_body)

    acc_v[...] = acc
    pltpu.sync_copy(acc_v, out_hbm.at[wid // 8, pl.ds((wid % 8) * _L, _L)])


def _tc_finish(p_ref, o_ref):
    o_ref[0, 0] = jnp.sum(p_ref[...]) * (_LAMBDA / _B)


_finish_call = pl.pallas_call(
    _tc_finish,
    out_shape=jax.ShapeDtypeStruct((1, 1), jnp.float32),
    out_specs=pl.BlockSpec(memory_space=pltpu.SMEM),
)


@jax.jit
def kernel(features, labels, centers):
    partials = _sc_partial_sums(features, labels.astype(jnp.int32), centers)
    return _finish_call(partials)[0, 0]
